# baseline (speedup 1.0000x reference)
import jax
import jax.numpy as jnp
from jax.experimental import pallas as pl
from jax.experimental.pallas import tpu as pltpu

N = 10000
K = 32
D = 128
C = 1000
BLK = 200


def _fused_body(x0_ref, x1_ref, ws_ref, wn_ref, b_ref, fcw_ref, fcb_ref,
                out_ref, scores_ref):
    x0b = x0_ref[...]
    neigh = x1_ref[...].reshape(BLK, K, D)
    mean = jnp.mean(neigh, axis=1)
    out = (
        jnp.dot(x0b, ws_ref[...], preferred_element_type=jnp.float32)
        + jnp.dot(mean, wn_ref[...], preferred_element_type=jnp.float32)
        + b_ref[...]
        + x0b
    )
    out_ref[...] = out
    hidden = jnp.maximum(out, 0.0)
    scores_ref[...] = (
        jnp.dot(hidden, fcw_ref[...], preferred_element_type=jnp.float32)
        + fcb_ref[...]
    )


def kernel(x0, x1, W_self, W_neigh, b, fc_W, fc_b):
    grid = (N // BLK,)
    b2 = b.reshape(1, D)
    fcb2 = fc_b.reshape(1, C)
    out, scores = pl.pallas_call(
        _fused_body,
        grid=grid,
        in_specs=[
            pl.BlockSpec((BLK, D), lambda i: (i, 0)),
            pl.BlockSpec((BLK * K, D), lambda i: (i, 0)),
            pl.BlockSpec((D, D), lambda i: (0, 0)),
            pl.BlockSpec((D, D), lambda i: (0, 0)),
            pl.BlockSpec((1, D), lambda i: (0, 0)),
            pl.BlockSpec((D, C), lambda i: (0, 0)),
            pl.BlockSpec((1, C), lambda i: (0, 0)),
        ],
        out_specs=[
            pl.BlockSpec((BLK, D), lambda i: (i, 0)),
            pl.BlockSpec((BLK, C), lambda i: (i, 0)),
        ],
        out_shape=[
            jax.ShapeDtypeStruct((N, D), jnp.float32),
            jax.ShapeDtypeStruct((N, C), jnp.float32),
        ],
        compiler_params=pltpu.CompilerParams(
            dimension_semantics=("arbitrary",),
        ),
    )(x0, x1, W_self, W_neigh, b2, fc_W, fcb2)
    return (out, scores)


# speedup vs baseline: 1.1250x; 1.1250x over previous
import jax
import jax.numpy as jnp
from jax.experimental import pallas as pl
from jax.experimental.pallas import tpu as pltpu

N = 10000
K = 32
D = 128
C = 1000
BLK = 1000


def _fused_body(x0_ref, x1_ref, ws_ref, wn_ref, b_ref, fcw_ref, fcb_ref,
                out_ref, scores_ref):
    x0b = x0_ref[...]
    neigh = x1_ref[...].reshape(BLK, K, D)
    mean = jnp.mean(neigh, axis=1)
    out = (
        jnp.dot(x0b, ws_ref[...], preferred_element_type=jnp.float32)
        + jnp.dot(mean, wn_ref[...], preferred_element_type=jnp.float32)
        + b_ref[...]
        + x0b
    )
    out_ref[...] = out
    hidden = jnp.maximum(out, 0.0)
    scores_ref[...] = (
        jnp.dot(hidden, fcw_ref[...], preferred_element_type=jnp.float32)
        + fcb_ref[...]
    )


def kernel(x0, x1, W_self, W_neigh, b, fc_W, fc_b):
    grid = (N // BLK,)
    b2 = b.reshape(1, D)
    fcb2 = fc_b.reshape(1, C)
    out, scores = pl.pallas_call(
        _fused_body,
        grid=grid,
        in_specs=[
            pl.BlockSpec((BLK, D), lambda i: (i, 0)),
            pl.BlockSpec((BLK * K, D), lambda i: (i, 0)),
            pl.BlockSpec((D, D), lambda i: (0, 0)),
            pl.BlockSpec((D, D), lambda i: (0, 0)),
            pl.BlockSpec((1, D), lambda i: (0, 0)),
            pl.BlockSpec((D, C), lambda i: (0, 0)),
            pl.BlockSpec((1, C), lambda i: (0, 0)),
        ],
        out_specs=[
            pl.BlockSpec((BLK, D), lambda i: (i, 0)),
            pl.BlockSpec((BLK, C), lambda i: (i, 0)),
        ],
        out_shape=[
            jax.ShapeDtypeStruct((N, D), jnp.float32),
            jax.ShapeDtypeStruct((N, C), jnp.float32),
        ],
        compiler_params=pltpu.CompilerParams(
            dimension_semantics=("arbitrary",),
        ),
    )(x0, x1, W_self, W_neigh, b2, fc_W, fcb2)
    return (out, scores)
